# baseline (speedup 1.0000x reference)
import functools

import jax
import jax.numpy as jnp
from jax import lax
from jax.experimental import pallas as pl
from jax.experimental.pallas import tpu as pltpu
from jax.experimental.pallas import tpu_sc as plsc

B = 16384
WORD = 26
NF = 13
ROWS = WORD + NF
D = 64
NC, NS, L = 2, 16, 16
NW = NC * NS
G = 128
NBLK = B // G
CPW = NBLK // NW
OUT_ROWS = ROWS * D * NBLK // 8 * 8
NBUF = 4
LEAD = 2

_mesh = plsc.VectorSubcoreMesh(core_axis_name="c", subcore_axis_name="s")


V = 1000000
NFULL = V // G


@functools.partial(
    pl.kernel,
    out_type=jax.ShapeDtypeStruct((V * D,), jnp.float32),
    mesh=_mesh,
    scratch_types=[
        pltpu.VMEM((L, L), jnp.int32),
        pltpu.VMEM((L, L), jnp.int32),
        [pltpu.VMEM((D, G), jnp.float32) for _ in range(2)],
        [pltpu.VMEM((D * G,), jnp.float32) for _ in range(2)],
        [pltpu.SemaphoreType.DMA for _ in range(2)],
        [pltpu.SemaphoreType.DMA for _ in range(2)],
    ],
    compiler_params=pltpu.CompilerParams(needs_layout_passes=False),
)
def _repack_kernel(lutt_hbm, perm_hbm, pre_hbm, tail_hbm, out_hbm, perm_v,
                   pre_v, slabs, tslabs, gsems, ssems):
    wid = lax.axis_index("s") * NC + lax.axis_index("c")
    pltpu.sync_copy(perm_hbm, perm_v)
    pltpu.sync_copy(pre_hbm, pre_v)
    perm = [perm_v[k, :] for k in range(L)]
    pre = [pre_v[k, :] for k in range(L)]

    def start_g(i, slot):
        c0 = pl.multiple_of((i * NW + wid) * G, 128)
        pltpu.async_copy(lutt_hbm.at[:, pl.ds(c0, G)], slabs[slot],
                         gsems[slot])

    def wait_g(slot):
        pltpu.make_async_copy(lutt_hbm.at[:, pl.ds(0, G)], slabs[slot],
                              gsems[slot]).wait()

    def start_s(i, slot):
        o0 = pl.multiple_of((i * NW + wid) * G * D, 8)
        pltpu.async_copy(tslabs[slot], out_hbm.at[pl.ds(o0, G * D)],
                         ssems[slot])

    def wait_s(slot):
        pltpu.make_async_copy(tslabs[slot], out_hbm.at[pl.ds(0, G * D)],
                              ssems[slot]).wait()

    lvec = perm[0]

    start_g(0, 0)

    def outer(i2, carry):
        for b in range(2):
            i = i2 * 2 + b
            cc = i * NW + wid
            nxt = (b + 1) % 2

            @pl.when((i + 1) * NW + wid < NFULL)
            def _():
                start_g(i + 1, nxt)

            @pl.when(cc < NFULL)
            def _():
                wait_g(b)

                @pl.when(i >= 2)
                def _():
                    wait_s(b)

                slab, tslab = slabs[b], tslabs[b]
                for i0 in range(0, G, L):
                    for d0 in range(0, D, L):
                        for k in range(L):
                            v = plsc.load_gather(
                                slab, [perm[k] + d0, lvec + i0])
                            plsc.store_scatter(
                                tslab, [pre[k] + (i0 * D + d0)], v)
                start_s(i, b)

            @pl.when(cc == NFULL)
            def _():
                pltpu.sync_copy(tail_hbm,
                                out_hbm.at[pl.ds(NFULL * G * D, D * D)])
        return carry

    lax.fori_loop(0, 123, outer, 0)
    for b in range(2):
        wait_s(b)


@functools.partial(
    pl.kernel,
    out_type=jax.ShapeDtypeStruct((ROWS * 8 * NBLK * 8, G), jnp.float32),
    mesh=_mesh,
    scratch_types=[
        pltpu.VMEM((WORD, CPW, G), jnp.int32),
        pltpu.VMEM((NF, CPW, G), jnp.float32),
        pltpu.VMEM((D, L), jnp.float32),
        pltpu.VMEM((D, L), jnp.float32),
        pltpu.VMEM((G // L, L), jnp.int32),
        pltpu.VMEM((D // L, L), jnp.int32),
        pltpu.VMEM((L, L), jnp.int32),
        [pltpu.VMEM((G, D), jnp.float32) for _ in range(NBUF)],
        [pltpu.VMEM((D, G), jnp.float32) for _ in range(NBUF)],
        [pltpu.VMEM((D,), jnp.int32) for _ in range(NBUF)],
        [pltpu.SemaphoreType.DMA for _ in range(NBUF)],
        [pltpu.SemaphoreType.DMA for _ in range(NBUF)],
    ],
    compiler_params=pltpu.CompilerParams(use_tc_tiling_on_sc=False,
                                        needs_layout_passes=False),
)
def _emb_kernel(idx_hbm, xnum_hbm, wsp_hbm, bsp_hbm, rq_hbm, pat_hbm,
                perm_hbm, lut_hbm, out_hbm, idx_v, xnum_v, w_v, b_v, rq_v,
                pat_v, perm_v, gbufs, tbufs, dstbs, gsems, ssems):
    wid = lax.axis_index("s") * NC + lax.axis_index("c")
    c0 = wid * CPW
    c0a = pl.multiple_of(wid * CPW, 4)
    pltpu.sync_copy(idx_hbm.at[:, pl.ds(c0a, CPW)], idx_v)
    pltpu.sync_copy(xnum_hbm.at[:, pl.ds(c0a, CPW)], xnum_v)
    pltpu.sync_copy(wsp_hbm, w_v)
    pltpu.sync_copy(bsp_hbm, b_v)
    pltpu.sync_copy(rq_hbm, rq_v)
    pltpu.sync_copy(pat_hbm, pat_v)
    pltpu.sync_copy(perm_hbm, perm_v)

    lvec = rq_v[0, :]
    pat = [pat_v[c, :] for c in range(D // L)]
    perm = [perm_v[k, :] for k in range(L)]

    def start_gather(j, cb, slot):
        pltpu.async_copy(lut_hbm.at[idx_v.at[j, cb]], gbufs[slot],
                         gsems[slot])

    def wait_gather(slot):
        pltpu.make_async_copy(lut_hbm.at[idx_v.at[0, 0]], gbufs[slot],
                              gsems[slot]).wait()

    def start_scatter(slot):
        pltpu.async_copy(tbufs[slot], out_hbm.at[dstbs[slot]], ssems[slot])

    def wait_scatter(slot):
        pltpu.make_async_copy(tbufs[slot], out_hbm.at[dstbs[0]],
                              ssems[slot]).wait()

    def set_dst(slot, base):
        for c in range(D // L):
            dstbs[slot][pl.ds(c * L, L)] = pat[c] + base

    for t in range(LEAD):
        start_gather(t // CPW, t % CPW, t % NBUF)

    def cat_outer(j, carry):
        t0 = j * CPW
        for cb in range(CPW):
            t = t0 + cb
            gslot = (cb + LEAD) % NBUF
            tg = t + LEAD
            if (CPW * (WORD - 1) + cb + LEAD) < CPW * WORD:
                start_gather(tg // CPW, tg % CPW, gslot)
            else:

                @pl.when(j < WORD - 1)
                def _():
                    start_gather((t0 + cb + LEAD) // CPW,
                                 (cb + LEAD) % CPW, gslot)

            wait_gather(cb)

            @pl.when(j >= 1)
            def _():
                wait_scatter(cb)

            gbuf, tbuf = gbufs[cb], tbufs[cb]

            def transpose_blk(r16, c2):
                rowv = lvec + r16 * L
                for c0 in range(0, D, L):
                    colvs = [perm[k] + c0 for k in range(L)]
                    vs = [plsc.load_gather(gbuf, [rowv, colvs[k]])
                          for k in range(L)]
                    for k in range(L):
                        plsc.store_scatter(tbuf, [colvs[k], rowv],
                                           vs[k] * 8.0)
                return c2

            lax.fori_loop(0, G // L, transpose_blk, 0)
            set_dst(cb, j * 8192 + (c0 + cb) * 8)
            start_scatter(cb)
        return carry

    lax.fori_loop(0, WORD, cat_outer, 0)
    for b in range(NBUF):
        wait_scatter(b)

    def num_outer(k, carry):
        for cb in range(CPW):

            @pl.when(k >= 1)
            def _():
                wait_scatter(cb)

            tbuf = tbufs[cb]
            xv = [xnum_v[k, cb, pl.ds(q * L, L)] for q in range(G // L)]

            def fill_d(d, c2):
                wvec = w_v[d, :]
                bvec = b_v[d, :]
                for q in range(G // L):
                    tbuf[d, pl.ds(q * L, L)] = xv[q] * wvec + bvec
                return c2

            lax.fori_loop(0, D, fill_d, 0)
            set_dst(cb, (WORD + k) * 8192 + (c0 + cb) * 8)
            start_scatter(cb)
        return carry

    lax.fori_loop(0, NF, num_outer, 0)
    for b in range(NBUF):
        wait_scatter(b)


def kernel(x, lut, W, b):
    idx = x[:, :WORD].astype(jnp.int32).T.reshape(WORD, NBLK, G)
    xnum = x[:, WORD:].T.reshape(NF, NBLK, G)
    wsp = jnp.broadcast_to((W[0] * 8.0)[:, None], (D, L))
    bsp = jnp.broadcast_to((b * 8.0)[:, None], (D, L))
    rq = jnp.arange(G, dtype=jnp.int32).reshape(G // L, L)
    dd = jnp.arange(D, dtype=jnp.int32)
    patt = ((dd // 8) * 1024 + dd % 8).reshape(D // L, L)
    ll = jnp.arange(L, dtype=jnp.int32)
    perm = (ll[None, :] + ll[:, None]) % L
    pre = ll[None, :] * D + perm
    tail = lut[NFULL * G:].reshape(-1)
    lut_rm = _repack_kernel(lut.T, perm, pre, tail).reshape(V, D)
    out_lin = _emb_kernel(idx, xnum, wsp, bsp, rq, patt, perm, lut_rm)
    out5 = out_lin.reshape(ROWS, 8, NBLK, 8, G)
    return out5.transpose(0, 1, 3, 2, 4).reshape(ROWS, D, B).transpose(2, 0, 1)


# speedup vs baseline: 1.8026x; 1.8026x over previous
import functools

import jax
import jax.numpy as jnp
from jax import lax
from jax.experimental import pallas as pl
from jax.experimental.pallas import tpu as pltpu
from jax.experimental.pallas import tpu_sc as plsc

B = 16384
WORD = 26
NF = 13
ROWS = WORD + NF
D = 64
NC, NS, L = 2, 16, 16
NW = NC * NS
G = 128
NBLK = B // G
CPW = NBLK // NW
OUT_ROWS = ROWS * D * NBLK // 8 * 8
NBUF = 4
LEAD = 2

_mesh = plsc.VectorSubcoreMesh(core_axis_name="c", subcore_axis_name="s")


@functools.partial(
    pl.kernel,
    out_type=jax.ShapeDtypeStruct((ROWS * 8 * NBLK * 8, G), jnp.float32),
    mesh=_mesh,
    scratch_types=[
        pltpu.VMEM((WORD, CPW, G), jnp.int32),
        pltpu.VMEM((NF, CPW, G), jnp.float32),
        pltpu.VMEM((D, L), jnp.float32),
        pltpu.VMEM((D, L), jnp.float32),
        pltpu.VMEM((G // L, L), jnp.int32),
        pltpu.VMEM((D // L, L), jnp.int32),
        pltpu.VMEM((L, L), jnp.int32),
        [pltpu.VMEM((G, 2 * D), jnp.float32) for _ in range(NBUF)],
        [pltpu.VMEM((D, G), jnp.float32) for _ in range(NBUF)],
        [pltpu.VMEM((D,), jnp.int32) for _ in range(NBUF)],
        [pltpu.SemaphoreType.DMA for _ in range(NBUF)],
        [pltpu.SemaphoreType.DMA for _ in range(NBUF)],
    ],
    compiler_params=pltpu.CompilerParams(use_tc_tiling_on_sc=False,
                                        needs_layout_passes=False),
)
def _emb_kernel(idx_hbm, xnum_hbm, wsp_hbm, bsp_hbm, rq_hbm, pat_hbm,
                perm_hbm, lut_hbm, out_hbm, idx_v, xnum_v, w_v, b_v, rq_v,
                pat_v, perm_v, gbufs, tbufs, dstbs, gsems, ssems):
    wid = lax.axis_index("s") * NC + lax.axis_index("c")
    c0 = wid * CPW
    c0a = pl.multiple_of(wid * CPW, 4)
    pltpu.sync_copy(idx_hbm.at[:, pl.ds(c0a, CPW)], idx_v)
    pltpu.sync_copy(xnum_hbm.at[:, pl.ds(c0a, CPW)], xnum_v)
    pltpu.sync_copy(wsp_hbm, w_v)
    pltpu.sync_copy(bsp_hbm, b_v)
    pltpu.sync_copy(rq_hbm, rq_v)
    pltpu.sync_copy(pat_hbm, pat_v)
    pltpu.sync_copy(perm_hbm, perm_v)

    lvec = rq_v[0, :]
    pat = [pat_v[c, :] for c in range(D // L)]
    perm = [perm_v[k, :] for k in range(L)]

    def start_gather(j, cb, slot):
        pltpu.async_copy(lut_hbm.at[idx_v.at[j, cb]], gbufs[slot],
                         gsems[slot])

    def wait_gather(slot):
        pltpu.make_async_copy(lut_hbm.at[idx_v.at[0, 0]], gbufs[slot],
                              gsems[slot]).wait()

    def start_scatter(slot):
        pltpu.async_copy(tbufs[slot], out_hbm.at[dstbs[slot]], ssems[slot])

    def wait_scatter(slot):
        pltpu.make_async_copy(tbufs[slot], out_hbm.at[dstbs[0]],
                              ssems[slot]).wait()

    def set_dst(slot, base):
        for c in range(D // L):
            dstbs[slot][pl.ds(c * L, L)] = pat[c] + base

    for t in range(LEAD):
        start_gather(t // CPW, t % CPW, t % NBUF)

    def cat_outer(j, carry):
        t0 = j * CPW
        for cb in range(CPW):
            t = t0 + cb
            gslot = (cb + LEAD) % NBUF
            tg = t + LEAD
            if (CPW * (WORD - 1) + cb + LEAD) < CPW * WORD:
                start_gather(tg // CPW, tg % CPW, gslot)
            else:

                @pl.when(j < WORD - 1)
                def _():
                    start_gather((t0 + cb + LEAD) // CPW,
                                 (cb + LEAD) % CPW, gslot)

            wait_gather(cb)

            @pl.when(j >= 1)
            def _():
                wait_scatter(cb)

            gbuf, tbuf = gbufs[cb], tbufs[cb]

            def transpose_blk(r16, c2):
                rowv = lvec + r16 * L
                for c0 in range(0, D, L):
                    colvs = [perm[k] + c0 for k in range(L)]
                    vs = [plsc.load_gather(gbuf, [rowv, colvs[k]])
                          for k in range(L)]
                    for k in range(L):
                        plsc.store_scatter(tbuf, [colvs[k], rowv],
                                           vs[k] * 8.0)
                return c2

            lax.fori_loop(0, G // L, transpose_blk, 0)
            set_dst(cb, j * 8192 + (c0 + cb) * 8)
            start_scatter(cb)
        return carry

    lax.fori_loop(0, WORD, cat_outer, 0)
    for b in range(NBUF):
        wait_scatter(b)

    def num_outer(k, carry):
        for cb in range(CPW):

            @pl.when(k >= 1)
            def _():
                wait_scatter(cb)

            tbuf = tbufs[cb]
            xv = [xnum_v[k, cb, pl.ds(q * L, L)] for q in range(G // L)]

            def fill_d(d, c2):
                wvec = w_v[d, :]
                bvec = b_v[d, :]
                for q in range(G // L):
                    tbuf[d, pl.ds(q * L, L)] = xv[q] * wvec + bvec
                return c2

            lax.fori_loop(0, D, fill_d, 0)
            set_dst(cb, (WORD + k) * 8192 + (c0 + cb) * 8)
            start_scatter(cb)
        return carry

    lax.fori_loop(0, NF, num_outer, 0)
    for b in range(NBUF):
        wait_scatter(b)


def kernel(x, lut, W, b):
    idx = x[:, :WORD].astype(jnp.int32).T.reshape(WORD, NBLK, G)
    xnum = x[:, WORD:].T.reshape(NF, NBLK, G)
    wsp = jnp.broadcast_to((W[0] * 8.0)[:, None], (D, L))
    bsp = jnp.broadcast_to((b * 8.0)[:, None], (D, L))
    rq = jnp.arange(G, dtype=jnp.int32).reshape(G // L, L)
    dd = jnp.arange(D, dtype=jnp.int32)
    patt = ((dd // 8) * 1024 + dd % 8).reshape(D // L, L)
    ll = jnp.arange(L, dtype=jnp.int32)
    perm = (ll[None, :] + ll[:, None]) % L
    lutp = jnp.pad(lut, ((0, 0), (0, 64)))
    out_lin = _emb_kernel(idx, xnum, wsp, bsp, rq, patt, perm, lutp)
    out5 = out_lin.reshape(ROWS, 8, NBLK, 8, G)
    return out5.transpose(0, 1, 3, 2, 4).reshape(ROWS, D, B).transpose(2, 0, 1)
